# baseline (speedup 1.0000x reference)
import functools

import jax
import jax.numpy as jnp
from jax import lax
from jax.experimental import pallas as pl
from jax.experimental.pallas import tpu as pltpu
from jax.experimental.pallas import tpu_sc as plsc

_N_SPARSE = 26
_N_DENSE = 13
_VOCAB = 100000
_EMB = 16
_B = 16384
_FH = _N_SPARSE // 2
_SLOTS = 16
_PW = 8
_GW = _SLOTS * _PW
_ROWS_H = _B * _SLOTS
_VSEG = _VOCAB // 8

_NC, _NS = 2, 16
_NW = _NC * _NS
_RPW = _ROWS_H // _NW
_CHUNK = 1024
_NCHUNK = _RPW // _CHUNK


def _sc_gather(tables_half, idx_flat):
  mesh = plsc.VectorSubcoreMesh(core_axis_name="c", subcore_axis_name="s")

  @functools.partial(
      pl.kernel,
      mesh=mesh,
      out_type=jax.ShapeDtypeStruct((_ROWS_H, _PW), jnp.int32),
      scratch_types=[
          pltpu.VMEM((_CHUNK,), jnp.int32),
          pltpu.VMEM((_CHUNK, _PW), jnp.int32),
          pltpu.SemaphoreType.DMA,
      ],
      compiler_params=pltpu.CompilerParams(use_tc_tiling_on_sc=False),
  )
  def k(tab_hbm, idx_hbm, out_hbm, idx_v, rows_v, sem):
    wid = lax.axis_index("s") * _NC + lax.axis_index("c")
    base = wid * _RPW
    for j in range(_NCHUNK):
      off = base + j * _CHUNK
      pltpu.sync_copy(idx_hbm.at[pl.ds(off, _CHUNK)], idx_v)
      pltpu.make_async_copy(tab_hbm.at[idx_v], rows_v, sem).start()
      pltpu.make_async_copy(tab_hbm.at[idx_v], rows_v, sem).wait()
      pltpu.sync_copy(rows_v, out_hbm.at[pl.ds(off, _CHUNK)])

  return k(tables_half, idx_flat)


def _tr_body(in_ref, out_hbm, x_scr, y_scr, sem):
  f = pl.program_id(0)
  nf = pl.num_programs(0)
  for j in range(8):
    x_scr[j * _EMB:(j + 1) * _EMB, :] = in_ref[0, :, j * _VSEG:(j + 1) * _VSEG]
  packs = []
  for j in range(8):
    lo = x_scr[j * _EMB:j * _EMB + _PW, :]
    hi = x_scr[j * _EMB + _PW:(j + 1) * _EMB, :]
    ulo = jax.lax.shift_right_logical(
        jax.lax.bitcast_convert_type(lo, jnp.int32), 16)
    uhi = jax.lax.bitwise_and(
        jax.lax.bitcast_convert_type(hi, jnp.int32),
        jnp.int32(-65536))
    packs.append(jax.lax.bitwise_or(uhi, ulo))
  p = jnp.concatenate(packs, axis=0)
  off = jax.lax.rem(f, 2) * _VSEG

  @pl.when(f >= 2)
  def _wait_slot():
    pltpu.make_async_copy(
        y_scr.at[pl.ds(off, _VSEG)],
        out_hbm.at[pl.ds((f - 2) * _VSEG, _VSEG)], sem).wait()

  y_scr[pl.ds(off, _VSEG), :] = p.T
  pltpu.make_async_copy(
      y_scr.at[pl.ds(off, _VSEG)],
      out_hbm.at[pl.ds(f * _VSEG, _VSEG)], sem).start()

  @pl.when(f == nf - 1)
  def _drain_all():
    for _ in range(2):
      pltpu.make_async_copy(
          y_scr.at[pl.ds(off, _VSEG)],
          out_hbm.at[pl.ds(f * _VSEG, _VSEG)], sem).wait()


def _tc_transpose(tphys):
  return pl.pallas_call(
      _tr_body,
      grid=(_FH,),
      in_specs=[pl.BlockSpec((1, _EMB, _VOCAB), lambda f: (f, 0, 0))],
      out_specs=pl.BlockSpec(memory_space=pl.ANY),
      out_shape=jax.ShapeDtypeStruct((_FH * _VSEG, 64), jnp.int32),
      scratch_shapes=[
          pltpu.VMEM((128, _VSEG), jnp.float32),
          pltpu.VMEM((2 * _VSEG, 64), jnp.int32),
          pltpu.SemaphoreType.DMA,
      ],
      compiler_params=pltpu.CompilerParams(
          dimension_semantics=("arbitrary",)),
  )(tphys)


def _dnn_body(g1_ref, g2_ref, d_ref, slo_ref, shi_ref, m_ref, w1_ref,
              w1b_ref, b1_ref, w2_ref, b2_ref, w3_ref, b3_ref, wf_ref,
              bf_ref, out_ref):
  f32 = jnp.float32
  dd = d_ref[...]
  m = m_ref[...]

  def unpack(g):
    lo = jax.lax.bitcast_convert_type(
        jax.lax.shift_left(g, 16), f32)
    hi = jax.lax.bitcast_convert_type(
        jax.lax.bitwise_and(g, jnp.int32(-65536)), f32)
    return lo, hi

  g1lo, g1hi = unpack(g1_ref[...])
  g2lo, g2hi = unpack(g2_ref[...])
  slo = slo_ref[...]
  shi = shi_ref[...]
  sum_e = (lax.dot(g1lo, slo, preferred_element_type=f32)
           + lax.dot(g1hi, shi, preferred_element_type=f32)
           + lax.dot(g2lo, slo, preferred_element_type=f32)
           + lax.dot(g2hi, shi, preferred_element_type=f32))
  t1 = jnp.sum(sum_e * sum_e, axis=1, keepdims=True)
  t2 = jnp.sum((g1lo * g1lo + g1hi * g1hi + g2lo * g2lo + g2hi * g2hi) * m,
               axis=1, keepdims=True)
  wide = 0.5 * (t1 - t2)
  w1 = w1_ref[...]
  h = lax.dot(g1lo, w1[0:128], preferred_element_type=f32)
  h = h + lax.dot(g1hi, w1[128:256], preferred_element_type=f32)
  h = h + lax.dot(g2lo, w1[256:384], preferred_element_type=f32)
  h = h + lax.dot(g2hi, w1[384:512], preferred_element_type=f32)
  h = h + lax.dot(dd, w1b_ref[...], preferred_element_type=f32)
  h = jax.nn.relu(h + b1_ref[...])
  h = jax.nn.relu(lax.dot(h, w2_ref[...], preferred_element_type=f32)
                  + b2_ref[...])
  h = jax.nn.relu(lax.dot(h, w3_ref[...], preferred_element_type=f32)
                  + b3_ref[...])
  z = lax.dot(wide + h, wf_ref[...], preferred_element_type=f32) + bf_ref[...]
  out_ref[...] = jax.nn.sigmoid(z)


_BB = 512


def _dnn(g1, g2, dense, slo, shi, m, w1, w1b, b1, w2, b2, w3, b3, wf, bf):
  def row_block(i):
    return (i, 0)

  def full(i):
    return (0, 0)

  return pl.pallas_call(
      _dnn_body,
      grid=(_B // _BB,),
      in_specs=[
          pl.BlockSpec((_BB, _GW), row_block),
          pl.BlockSpec((_BB, _GW), row_block),
          pl.BlockSpec((_BB, _N_DENSE), row_block),
          pl.BlockSpec((_GW, _EMB), full),
          pl.BlockSpec((_GW, _EMB), full),
          pl.BlockSpec((1, _GW), full),
          pl.BlockSpec((4 * _GW, 256), full),
          pl.BlockSpec((_N_DENSE, 256), full),
          pl.BlockSpec((1, 256), full),
          pl.BlockSpec((256, 128), full),
          pl.BlockSpec((1, 128), full),
          pl.BlockSpec((128, 64), full),
          pl.BlockSpec((1, 64), full),
          pl.BlockSpec((64, 1), full),
          pl.BlockSpec((1, 1), full),
      ],
      out_specs=pl.BlockSpec((_BB, 1), row_block),
      out_shape=jax.ShapeDtypeStruct((_B, 1), jnp.float32),
      compiler_params=pltpu.CompilerParams(
          dimension_semantics=("parallel",)),
  )(g1, g2, dense, slo, shi, m, w1, w1b, b1, w2, b2, w3, b3, wf, bf)


def _half_idx(sparse_half):
  perm = (jnp.arange(_FH, dtype=jnp.int32)[None, :] * _VSEG
          + sparse_half % _VSEG) * 8 + sparse_half // _VSEG
  idx = jnp.concatenate([perm, perm[:, :_SLOTS - _FH]], axis=1)
  return idx.reshape(-1)


def _w1_group(w1v, half, dgrp):
  rows = w1v[half * _FH:(half + 1) * _FH, dgrp * _PW:(dgrp + 1) * _PW]
  rows = rows.reshape(_FH * _PW, 256)
  return jnp.concatenate(
      [rows, jnp.zeros(((_SLOTS - _FH) * _PW, 256), jnp.float32)], axis=0)


def kernel(x, tables, W1, b1, W2, b2, W3, b3, Wf, bf):
  sparse_idx = x[:, :_N_SPARSE].astype(jnp.int32)
  dense = x[:, _N_SPARSE:]
  idx1 = _half_idx(sparse_idx[:, :_FH])
  idx2 = _half_idx(sparse_idx[:, _FH:])
  tp1 = jnp.transpose(tables[:_FH], (0, 2, 1))
  tp2 = jnp.transpose(tables[_FH:], (0, 2, 1))
  t1 = _tc_transpose(tp1)
  th1 = t1.reshape(_FH * _VOCAB, _PW)
  g1 = _sc_gather(th1, idx1).reshape(_B, _GW)
  t2 = _tc_transpose(tp2)
  th2 = t2.reshape(_FH * _VOCAB, _PW)
  g2 = _sc_gather(th2, idx2).reshape(_B, _GW)
  eye = jnp.eye(_PW, dtype=jnp.float32)
  slo_f = jnp.tile(eye, (_FH, 1))
  zsel = jnp.zeros(((_SLOTS - _FH) * _PW, _PW), jnp.float32)
  zw = jnp.zeros((_FH * _PW + (_SLOTS - _FH) * _PW, _PW), jnp.float32)
  slo = jnp.concatenate(
      [jnp.concatenate([slo_f, zsel], axis=0), zw], axis=1)
  shi = jnp.concatenate(
      [zw, jnp.concatenate([slo_f, zsel], axis=0)], axis=1)
  m = (jnp.arange(_GW, dtype=jnp.float32)
       < _FH * _PW).astype(jnp.float32)[None, :]
  w1v = W1[:_N_SPARSE * _EMB].reshape(_N_SPARSE, _EMB, 256)
  w1 = jnp.concatenate(
      [_w1_group(w1v, 0, 0), _w1_group(w1v, 0, 1),
       _w1_group(w1v, 1, 0), _w1_group(w1v, 1, 1)], axis=0)
  return _dnn(g1, g2, dense, slo, shi, m, w1, W1[_N_SPARSE * _EMB:],
              b1.reshape(1, -1), W2, b2.reshape(1, -1), W3,
              b3.reshape(1, -1), Wf, bf.reshape(1, 1))


# speedup vs baseline: 1.7062x; 1.7062x over previous
import functools

import jax
import jax.numpy as jnp
from jax import lax
from jax.experimental import pallas as pl
from jax.experimental.pallas import tpu as pltpu
from jax.experimental.pallas import tpu_sc as plsc

_N_SPARSE = 26
_N_DENSE = 13
_VOCAB = 100000
_EMB = 16
_B = 16384
_FLAT = _N_SPARSE * _EMB
_ROWS = _B * _N_SPARSE
_VSEG = _VOCAB // 8

_NC, _NS = 2, 16
_NW = _NC * _NS
_RPW = _ROWS // _NW
_CHUNK = 1024
_NCHUNK = _RPW // _CHUNK


def _sc_gather(tables_flat, idx_flat):
  mesh = plsc.VectorSubcoreMesh(core_axis_name="c", subcore_axis_name="s")

  @functools.partial(
      pl.kernel,
      mesh=mesh,
      out_type=jax.ShapeDtypeStruct((_ROWS, _EMB), jnp.float32),
      scratch_types=[
          pltpu.VMEM((_CHUNK,), jnp.int32),
          pltpu.VMEM((_CHUNK, _EMB), jnp.float32),
          pltpu.SemaphoreType.DMA,
      ],
      compiler_params=pltpu.CompilerParams(use_tc_tiling_on_sc=False),
  )
  def k(tab_hbm, idx_hbm, out_hbm, idx_v, rows_v, sem):
    wid = lax.axis_index("s") * _NC + lax.axis_index("c")
    base = wid * _RPW
    for j in range(_NCHUNK):
      off = base + j * _CHUNK
      pltpu.sync_copy(idx_hbm.at[pl.ds(off, _CHUNK)], idx_v)
      pltpu.make_async_copy(tab_hbm.at[idx_v], rows_v, sem).start()
      pltpu.make_async_copy(tab_hbm.at[idx_v], rows_v, sem).wait()
      pltpu.sync_copy(rows_v, out_hbm.at[pl.ds(off, _CHUNK)])

  return k(tables_flat, idx_flat)


def _tr_body(in_ref, out_hbm, x_scr, y_scr, sem):
  f = pl.program_id(0)
  nf = pl.num_programs(0)
  for j in range(8):
    x_scr[j * _EMB:(j + 1) * _EMB, :] = in_ref[0, :, j * _VSEG:(j + 1) * _VSEG]
  off = jax.lax.rem(f, 2) * _VSEG

  @pl.when(f >= 2)
  def _wait_slot():
    pltpu.make_async_copy(
        y_scr.at[pl.ds(off, _VSEG)],
        out_hbm.at[pl.ds((f - 2) * _VSEG, _VSEG)], sem).wait()

  y_scr[pl.ds(off, _VSEG), :] = x_scr[...].T
  pltpu.make_async_copy(
      y_scr.at[pl.ds(off, _VSEG)],
      out_hbm.at[pl.ds(f * _VSEG, _VSEG)], sem).start()

  @pl.when(f == nf - 1)
  def _drain_all():
    for _ in range(2):
      pltpu.make_async_copy(
          y_scr.at[pl.ds(off, _VSEG)],
          out_hbm.at[pl.ds(f * _VSEG, _VSEG)], sem).wait()


def _tc_transpose(tphys):
  rows_pf = _VOCAB * _EMB // 128
  return pl.pallas_call(
      _tr_body,
      grid=(_N_SPARSE,),
      in_specs=[pl.BlockSpec((1, _EMB, _VOCAB), lambda f: (f, 0, 0))],
      out_specs=pl.BlockSpec(memory_space=pl.ANY),
      out_shape=jax.ShapeDtypeStruct((_N_SPARSE * rows_pf, 128), jnp.float32),
      scratch_shapes=[
          pltpu.VMEM((128, _VSEG), jnp.float32),
          pltpu.VMEM((2 * _VSEG, 128), jnp.float32),
          pltpu.SemaphoreType.DMA,
      ],
      compiler_params=pltpu.CompilerParams(
          dimension_semantics=("arbitrary",)),
  )(tphys)


def _dnn_body(g_ref, d_ref, s_ref, w1a_ref, w1b_ref, b1_ref, w2_ref, b2_ref,
              w3_ref, b3_ref, wf_ref, bf_ref, out_ref):
  f32 = jnp.float32
  g = g_ref[...]
  dd = d_ref[...]
  sum_e = lax.dot(g, s_ref[...], preferred_element_type=f32)
  t1 = jnp.sum(sum_e * sum_e, axis=1, keepdims=True)
  t2 = jnp.sum(g * g, axis=1, keepdims=True)
  wide = 0.5 * (t1 - t2)
  h = lax.dot(g, w1a_ref[...], preferred_element_type=f32)
  h = h + lax.dot(dd, w1b_ref[...], preferred_element_type=f32)
  h = jax.nn.relu(h + b1_ref[...])
  h = jax.nn.relu(lax.dot(h, w2_ref[...], preferred_element_type=f32)
                  + b2_ref[...])
  h = jax.nn.relu(lax.dot(h, w3_ref[...], preferred_element_type=f32)
                  + b3_ref[...])
  z = lax.dot(wide + h, wf_ref[...], preferred_element_type=f32) + bf_ref[...]
  out_ref[...] = jax.nn.sigmoid(z)


_BB = 512


def _dnn(g, dense, s, w1a, w1b, b1, w2, b2, w3, b3, wf, bf):
  def row_block(i):
    return (i, 0)

  def full(i):
    return (0, 0)

  return pl.pallas_call(
      _dnn_body,
      grid=(_B // _BB,),
      in_specs=[
          pl.BlockSpec((_BB, _FLAT), row_block),
          pl.BlockSpec((_BB, _N_DENSE), row_block),
          pl.BlockSpec((_FLAT, _EMB), full),
          pl.BlockSpec((_FLAT, 256), full),
          pl.BlockSpec((_N_DENSE, 256), full),
          pl.BlockSpec((1, 256), full),
          pl.BlockSpec((256, 128), full),
          pl.BlockSpec((1, 128), full),
          pl.BlockSpec((128, 64), full),
          pl.BlockSpec((1, 64), full),
          pl.BlockSpec((64, 1), full),
          pl.BlockSpec((1, 1), full),
      ],
      out_specs=pl.BlockSpec((_BB, 1), row_block),
      out_shape=jax.ShapeDtypeStruct((_B, 1), jnp.float32),
      compiler_params=pltpu.CompilerParams(
          dimension_semantics=("parallel",)),
  )(g, dense, s, w1a, w1b, b1, w2, b2, w3, b3, wf, bf)


def kernel(x, tables, W1, b1, W2, b2, W3, b3, Wf, bf):
  sparse_idx = x[:, :_N_SPARSE].astype(jnp.int32)
  dense = x[:, _N_SPARSE:]
  perm_row = (jnp.arange(_N_SPARSE, dtype=jnp.int32)[None, :] * _VSEG
              + sparse_idx % _VSEG) * 8 + sparse_idx // _VSEG
  idx_flat = perm_row.reshape(-1)
  tphys = jnp.transpose(tables, (0, 2, 1))
  t128 = _tc_transpose(tphys)
  tables_flat = t128.reshape(_N_SPARSE * _VOCAB, _EMB)
  gathered = _sc_gather(tables_flat, idx_flat)
  g = gathered.reshape(_B, _FLAT)
  s = jnp.tile(jnp.eye(_EMB, dtype=jnp.float32), (_N_SPARSE, 1))
  return _dnn(g, dense, s, W1[:_FLAT], W1[_FLAT:], b1.reshape(1, -1),
              W2, b2.reshape(1, -1), W3, b3.reshape(1, -1),
              Wf, bf.reshape(1, 1))
